# baseline (speedup 1.0000x reference)
import jax
import jax.numpy as jnp
from jax import lax
from jax.experimental import pallas as pl
from jax.experimental.pallas import tpu as pltpu

B, S, D, O = 8, 2048, 1024, 1024
BM = 512


def _mm_body(x_ref, w_ref, b_ref, m_ref, o_ref):
    xb = x_ref[...].astype(jnp.bfloat16)
    acc = lax.dot_general(
        xb, w_ref[...],
        dimension_numbers=(((1,), (0,)), ((), ())),
        preferred_element_type=jnp.float32,
    )
    o_ref[...] = (acc + b_ref[...]) * m_ref[...]


def kernel(input, data_mask, W, b):
    x = input.reshape(B * S, D)
    maskf = data_mask.reshape(B * S, 1).astype(jnp.float32)
    wtb = W.T.astype(jnp.bfloat16)
    b2 = b.reshape(1, O)

    grid = (B * S // BM,)
    out = pl.pallas_call(
        _mm_body,
        grid=grid,
        in_specs=[
            pl.BlockSpec((BM, D), lambda i: (i, 0)),
            pl.BlockSpec((D, O), lambda i: (0, 0)),
            pl.BlockSpec((1, O), lambda i: (0, 0)),
            pl.BlockSpec((BM, 1), lambda i: (i, 0)),
        ],
        out_specs=pl.BlockSpec((BM, O), lambda i: (i, 0)),
        out_shape=jax.ShapeDtypeStruct((B * S, O), jnp.float32),
        compiler_params=pltpu.CompilerParams(
            dimension_semantics=("parallel",),
        ),
    )(x, wtb, b2, maskf)
    return out.reshape(B, S, O)


# speedup vs baseline: 1.1469x; 1.1469x over previous
import jax
import jax.numpy as jnp
from jax import lax
from jax.experimental import pallas as pl
from jax.experimental.pallas import tpu as pltpu

B, S, D, O = 8, 2048, 1024, 1024
BM = 1024


def _mm_body(x_ref, w_ref, b_ref, m_ref, o_ref):
    xb = x_ref[...].astype(jnp.bfloat16)
    acc = lax.dot_general(
        xb, w_ref[...],
        dimension_numbers=(((1,), (0,)), ((), ())),
        preferred_element_type=jnp.float32,
    )
    o_ref[...] = (acc + b_ref[...]) * m_ref[...]


def kernel(input, data_mask, W, b):
    x = input.reshape(B * S, D)
    maskf = data_mask.reshape(B * S, 1).astype(jnp.float32)
    wtb = W.T.astype(jnp.bfloat16)
    b2 = b.reshape(1, O)

    grid = (B * S // BM,)
    out = pl.pallas_call(
        _mm_body,
        grid=grid,
        in_specs=[
            pl.BlockSpec((BM, D), lambda i: (i, 0)),
            pl.BlockSpec((D, O), lambda i: (0, 0)),
            pl.BlockSpec((1, O), lambda i: (0, 0)),
            pl.BlockSpec((BM, 1), lambda i: (i, 0)),
        ],
        out_specs=pl.BlockSpec((BM, O), lambda i: (i, 0)),
        out_shape=jax.ShapeDtypeStruct((B * S, O), jnp.float32),
        compiler_params=pltpu.CompilerParams(
            dimension_semantics=("parallel",),
        ),
    )(x, wtb, b2, maskf)
    return out.reshape(B, S, O)


# speedup vs baseline: 1.1904x; 1.0379x over previous
import jax
import jax.numpy as jnp
from jax import lax
from jax.experimental import pallas as pl
from jax.experimental.pallas import tpu as pltpu

B, S, D, O = 8, 2048, 1024, 1024
BM = 2048


def _mm_body(x_ref, w_ref, b_ref, m_ref, o_ref):
    xb = x_ref[...].astype(jnp.bfloat16)
    acc = lax.dot_general(
        xb, w_ref[...],
        dimension_numbers=(((1,), (0,)), ((), ())),
        preferred_element_type=jnp.float32,
    )
    o_ref[...] = (acc + b_ref[...]) * m_ref[...]


def kernel(input, data_mask, W, b):
    x = input.reshape(B * S, D)
    maskf = data_mask.reshape(B * S, 1).astype(jnp.float32)
    wtb = W.T.astype(jnp.bfloat16)
    b2 = b.reshape(1, O)

    grid = (B * S // BM,)
    out = pl.pallas_call(
        _mm_body,
        grid=grid,
        in_specs=[
            pl.BlockSpec((BM, D), lambda i: (i, 0)),
            pl.BlockSpec((D, O), lambda i: (0, 0)),
            pl.BlockSpec((1, O), lambda i: (0, 0)),
            pl.BlockSpec((BM, 1), lambda i: (i, 0)),
        ],
        out_specs=pl.BlockSpec((BM, O), lambda i: (i, 0)),
        out_shape=jax.ShapeDtypeStruct((B * S, O), jnp.float32),
        compiler_params=pltpu.CompilerParams(
            dimension_semantics=("parallel",),
        ),
    )(x, wtb, b2, maskf)
    return out.reshape(B, S, O)


# speedup vs baseline: 1.2714x; 1.0681x over previous
import jax
import jax.numpy as jnp
from jax import lax
from jax.experimental import pallas as pl
from jax.experimental.pallas import tpu as pltpu

B, S, D, O = 8, 2048, 1024, 1024
BM = 2048


def _mm_body(x_ref, w_ref, b_ref, m_ref, o_ref):
    xb = x_ref[...].astype(jnp.bfloat16)
    wb = w_ref[...].astype(jnp.bfloat16)
    acc = lax.dot_general(
        xb, wb,
        dimension_numbers=(((1,), (1,)), ((), ())),
        preferred_element_type=jnp.float32,
    )
    mf = m_ref[...].astype(jnp.float32)
    o_ref[...] = (acc + b_ref[...]) * mf


def kernel(input, data_mask, W, b):
    x = input.reshape(B * S, D)
    m2 = data_mask.reshape(B * S, 1)
    b2 = b.reshape(1, O)

    grid = (B * S // BM,)
    out = pl.pallas_call(
        _mm_body,
        grid=grid,
        in_specs=[
            pl.BlockSpec((BM, D), lambda i: (i, 0)),
            pl.BlockSpec((O, D), lambda i: (0, 0)),
            pl.BlockSpec((1, O), lambda i: (0, 0)),
            pl.BlockSpec((BM, 1), lambda i: (i, 0)),
        ],
        out_specs=pl.BlockSpec((BM, O), lambda i: (i, 0)),
        out_shape=jax.ShapeDtypeStruct((B * S, O), jnp.float32),
        compiler_params=pltpu.CompilerParams(
            dimension_semantics=("parallel",),
        ),
    )(x, W, b2, m2)
    return out.reshape(B, S, O)
